# baseline (speedup 1.0000x reference)
import functools

import jax
import jax.numpy as jnp
from jax import lax
from jax.experimental import pallas as pl
from jax.experimental.pallas import tpu as pltpu
from jax.experimental.pallas import tpu_sc as plsc

D = 32
NT = 16384
H = 50
NC = 2
NS = 16
NW = NC * NS
TPW = NT // NW
G = 8
NG = TPW // G


NR = 1000000
TC_ = 800
NCHK = NR // TC_
KMAX = (NCHK + NW - 1) // NW


def _make_transposer():
  mesh = plsc.VectorSubcoreMesh(core_axis_name="c", subcore_axis_name="s")

  @functools.partial(
      pl.kernel,
      mesh=mesh,
      compiler_params=pltpu.CompilerParams(
          use_tc_tiling_on_sc=False, needs_layout_passes=False
      ),
      out_type=jax.ShapeDtypeStruct((NR, D), jnp.float32),
      scratch_types=[
          pltpu.VMEM((D, TC_), jnp.float32),
          pltpu.VMEM((D, TC_), jnp.float32),
          pltpu.VMEM((TC_, D), jnp.float32),
          pltpu.VMEM((TC_, D), jnp.float32),
          pltpu.SemaphoreType.DMA,
          pltpu.SemaphoreType.DMA,
          pltpu.SemaphoreType.DMA,
          pltpu.SemaphoreType.DMA,
      ],
  )
  def transpose_kernel(tt_hbm, out_hbm, ib0, ib1, ob0, ob1,
                       rsem0, rsem1, wsem0, wsem1):
    wid = lax.axis_index("s") * NC + lax.axis_index("c")
    ibs, obs = (ib0, ib1), (ob0, ob1)
    rsems, wsems = (rsem0, rsem1), (wsem0, wsem1)
    lanes = lax.iota(jnp.int32, 16)

    def start_read(k, b):
      c = wid + NW * k
      pltpu.make_async_copy(
          tt_hbm.at[:, pl.ds(c * TC_, TC_)], ibs[b], rsems[b]
      ).start()

    def wait_read(b):
      pltpu.make_async_copy(
          tt_hbm.at[:, pl.ds(0, TC_)], ibs[b], rsems[b]
      ).wait()

    def transpose_chunk(ib, ob):
      def rows(r0):
        for rr in range(16):
          r = r0 + rr
          for fh in range(2):
            v = plsc.load_gather(
                ib, [lanes + 16 * fh, jnp.full((16,), r, jnp.int32)]
            )
            ob[r, pl.ds(16 * fh, 16)] = v
      pl.loop(0, TC_, step=16)(rows)

    start_read(0, 0)

    def round_pair(p):
      for b in range(2):
        k = p + b
        c = wid + NW * k

        @pl.when(c < NCHK)
        def _():
          wait_read(b)

          @pl.when(c + NW < NCHK)
          def _():
            start_read(k + 1, 1 - b)

          @pl.when(k >= 2)
          def _():
            pltpu.make_async_copy(
                obs[b], out_hbm.at[pl.ds(0, TC_)], wsems[b]
            ).wait()

          transpose_chunk(ibs[b], obs[b])
          pltpu.make_async_copy(
              obs[b], out_hbm.at[pl.ds(c * TC_, TC_)], wsems[b]
          ).start()

    pl.loop(0, KMAX, step=2)(round_pair)

    for b in range(2):
      pltpu.make_async_copy(
          obs[b], out_hbm.at[pl.ds(0, TC_)], wsems[b]
      ).wait()

  return transpose_kernel


def _make_gather():
  mesh = plsc.VectorSubcoreMesh(core_axis_name="c", subcore_axis_name="s")

  @functools.partial(
      pl.kernel,
      mesh=mesh,
      compiler_params=pltpu.CompilerParams(use_tc_tiling_on_sc=False),
      out_type=jax.ShapeDtypeStruct((NT, H, D), jnp.float32),
      scratch_types=[
          pltpu.VMEM((TPW, H), jnp.int32),
          pltpu.VMEM((G, H, D), jnp.float32),
          pltpu.VMEM((G, H, D), jnp.float32),
          pltpu.SemaphoreType.DMA,
          pltpu.SemaphoreType.DMA,
          pltpu.SemaphoreType.DMA,
          pltpu.SemaphoreType.DMA,
      ],
  )
  def gather_kernel(tokens_hbm, table_hbm, out_hbm, idx_v, buf0, buf1,
                    gsem0, gsem1, ssem0, ssem1):
    wid = lax.axis_index("s") * NC + lax.axis_index("c")
    base = wid * TPW
    pltpu.sync_copy(tokens_hbm.at[pl.ds(base, TPW)], idx_v)
    bufs = (buf0, buf1)
    gsems = (gsem0, gsem1)
    ssems = (ssem0, ssem1)

    def fire(g, buf, gsem):
      for j in range(G):
        pltpu.make_async_copy(
            table_hbm.at[idx_v.at[g * G + j]],
            buf.at[j],
            gsem,
        ).start()

    def drain_gathers(b):
      pltpu.make_async_copy(
          out_hbm.at[pl.ds(base, G)], bufs[b], gsems[b]
      ).wait()

    def pair(p):
      for b in range(2):
        g = p + b
        ob = 1 - b

        @pl.when(g >= 2)
        def _():
          pltpu.make_async_copy(
              bufs[b], out_hbm.at[pl.ds(base + (g - 2) * G, G)], ssems[b]
          ).wait()

        fire(g, bufs[b], gsems[b])

        @pl.when(g >= 1)
        def _():
          drain_gathers(ob)
          pltpu.make_async_copy(
              bufs[ob], out_hbm.at[pl.ds(base + (g - 1) * G, G)], ssems[ob]
          ).start()

    pl.loop(0, NG, step=2)(pair)

    drain_gathers(1)
    final = pltpu.make_async_copy(
        bufs[1], out_hbm.at[pl.ds(base + (NG - 1) * G, G)], ssems[1]
    )
    final.start()
    pltpu.make_async_copy(
        bufs[0], out_hbm.at[pl.ds(base + (NG - 2) * G, G)], ssems[0]
    ).wait()
    final.wait()

  return gather_kernel


_transposer = _make_transposer()
_gather = _make_gather()


def kernel(tokens, table):
  table_rm = _transposer(jnp.transpose(table))
  return _gather(tokens, table_rm)


# speedup vs baseline: 3.0631x; 3.0631x over previous
import functools

import jax
import jax.numpy as jnp
from jax import lax
from jax.experimental import pallas as pl
from jax.experimental.pallas import tpu as pltpu
from jax.experimental.pallas import tpu_sc as plsc

D = 32
NT = 16384
H = 50
NC = 2
NS = 16
NW = NC * NS
TPW = NT // NW
CH = 128
NQ = TPW // CH


def _make_gather():
  mesh = plsc.VectorSubcoreMesh(core_axis_name="c", subcore_axis_name="s")

  @functools.partial(
      pl.kernel,
      mesh=mesh,
      compiler_params=pltpu.CompilerParams(
          use_tc_tiling_on_sc=False, needs_layout_passes=False
      ),
      out_type=jax.ShapeDtypeStruct((H, D, NT), jnp.float32),
      scratch_types=[
          pltpu.VMEM((TPW, H), jnp.int32),
          pltpu.VMEM((H, NQ, CH), jnp.int32),
          pltpu.VMEM((CH, D), jnp.float32),
          pltpu.VMEM((CH, D), jnp.float32),
          pltpu.VMEM((D, TPW), jnp.float32),
          pltpu.VMEM((D, TPW), jnp.float32),
          pltpu.SemaphoreType.DMA,
          pltpu.SemaphoreType.DMA,
          pltpu.SemaphoreType.DMA,
          pltpu.SemaphoreType.DMA,
      ],
  )
  def gather_kernel(tokens_hbm, table_hbm, out_hbm, idx_v, idx_t,
                    rows0, rows1, tb0, tb1, gsem0, gsem1, ssem0, ssem1):
    wid = lax.axis_index("s") * NC + lax.axis_index("c")
    b0 = wid * TPW
    rows = (rows0, rows1)
    gsems = (gsem0, gsem1)
    tbs = (tb0, tb1)
    ssems = (ssem0, ssem1)
    lanes = lax.iota(jnp.int32, 16)

    pltpu.sync_copy(tokens_hbm.at[pl.ds(b0, TPW)], idx_v)

    def idx_rows(h):
      for g in range(TPW // 16):
        v = plsc.load_gather(
            idx_v, [g * 16 + lanes, jnp.full((16,), h, jnp.int32)]
        )
        idx_t[h, g // 8, pl.ds((g % 8) * 16, 16)] = v
    pl.loop(0, H)(idx_rows)

    def fire(h, q, b):
      pltpu.make_async_copy(
          table_hbm.at[idx_t.at[h, q]], rows[b], gsems[b]
      ).start()

    def wait_gather(b):
      pltpu.make_async_copy(
          table_hbm.at[pl.ds(0, CH)], rows[b], gsems[b]
      ).wait()

    def transpose_block(rb, tb, q):
      def sgroup(s0):
        for f in range(D):
          v = plsc.load_gather(
              rb, [s0 + lanes, jnp.full((16,), f, jnp.int32)]
          )
          tb[f, pl.ds(q * CH + s0, 16)] = v
      pl.loop(0, CH, step=16)(sgroup)

    fire(0, 0, 0)

    def h_pair(hp):
      for hb in range(2):
        h = hp + hb
        tb = tbs[hb]

        @pl.when(h >= 2)
        def _():
          pltpu.make_async_copy(
              tb, out_hbm.at[0, :, pl.ds(0, TPW)], ssems[hb]
          ).wait()

        for q in range(NQ):
          wait_gather(q % 2)
          if q < NQ - 1:
            fire(h, q + 1, (q + 1) % 2)
          else:
            @pl.when(h < H - 1)
            def _():
              fire(h + 1, 0, 0)
          transpose_block(rows[q % 2], tb, q)

        pltpu.make_async_copy(
            tb, out_hbm.at[h, :, pl.ds(b0, TPW)], ssems[hb]
        ).start()

    pl.loop(0, H, step=2)(h_pair)

    for hb in range(2):
      pltpu.make_async_copy(
          tbs[hb], out_hbm.at[0, :, pl.ds(0, TPW)], ssems[hb]
      ).wait()

  return gather_kernel


_gather = _make_gather()


def kernel(tokens, table):
  out_hmaj = _gather(tokens, table)
  return out_hmaj.transpose(2, 0, 1)


# speedup vs baseline: 3.7226x; 1.2153x over previous
import functools

import jax
import jax.numpy as jnp
from jax import lax
from jax.experimental import pallas as pl
from jax.experimental.pallas import tpu as pltpu
from jax.experimental.pallas import tpu_sc as plsc

D = 32
NT = 16384
H = 50
NC = 2
NS = 16
NW = NC * NS
TPW = NT // NW
G = 8
NG = TPW // G


def _make_gather():
  mesh = plsc.VectorSubcoreMesh(core_axis_name="c", subcore_axis_name="s")

  @functools.partial(
      pl.kernel,
      mesh=mesh,
      compiler_params=pltpu.CompilerParams(use_tc_tiling_on_sc=False),
      out_type=jax.ShapeDtypeStruct((NT, H, D), jnp.float32),
      scratch_types=[
          pltpu.VMEM((TPW, H), jnp.int32),
          pltpu.VMEM((G, H, D), jnp.float32),
          pltpu.VMEM((G, H, D), jnp.float32),
          pltpu.SemaphoreType.DMA,
          pltpu.SemaphoreType.DMA,
          pltpu.SemaphoreType.DMA,
          pltpu.SemaphoreType.DMA,
      ],
  )
  def gather_kernel(tokens_hbm, table_hbm, out_hbm, idx_v, buf0, buf1,
                    gsem0, gsem1, ssem0, ssem1):
    wid = lax.axis_index("s") * NC + lax.axis_index("c")
    base = wid * TPW
    pltpu.sync_copy(tokens_hbm.at[pl.ds(base, TPW)], idx_v)
    bufs = (buf0, buf1)
    gsems = (gsem0, gsem1)
    ssems = (ssem0, ssem1)

    def fire(g, buf, gsem):
      for j in range(G):
        pltpu.make_async_copy(
            table_hbm.at[idx_v.at[g * G + j]],
            buf.at[j],
            gsem,
        ).start()

    def drain_gathers(b):
      pltpu.make_async_copy(
          out_hbm.at[pl.ds(base, G)], bufs[b], gsems[b]
      ).wait()

    def pair(p):
      for b in range(2):
        g = p + b
        ob = 1 - b

        @pl.when(g >= 2)
        def _():
          pltpu.make_async_copy(
              bufs[b], out_hbm.at[pl.ds(base + (g - 2) * G, G)], ssems[b]
          ).wait()

        fire(g, bufs[b], gsems[b])

        @pl.when(g >= 1)
        def _():
          drain_gathers(ob)
          pltpu.make_async_copy(
              bufs[ob], out_hbm.at[pl.ds(base + (g - 1) * G, G)], ssems[ob]
          ).start()

    pl.loop(0, NG, step=2)(pair)

    drain_gathers(1)
    final = pltpu.make_async_copy(
        bufs[1], out_hbm.at[pl.ds(base + (NG - 1) * G, G)], ssems[1]
    )
    final.start()
    pltpu.make_async_copy(
        bufs[0], out_hbm.at[pl.ds(base + (NG - 2) * G, G)], ssems[0]
    ).wait()
    final.wait()

  return gather_kernel


_gather = _make_gather()


def kernel(tokens, table):
  return _gather(tokens, table)


# speedup vs baseline: 3.9583x; 1.0633x over previous
import functools

import jax
import jax.numpy as jnp
from jax import lax
from jax.experimental import pallas as pl
from jax.experimental.pallas import tpu as pltpu
from jax.experimental.pallas import tpu_sc as plsc

D = 32
NT = 16384
H = 50
NC = 2
NS = 16
NW = NC * NS
TPW = NT // NW
CH = 128
NQ = TPW // CH


def _make_gather():
  mesh = plsc.VectorSubcoreMesh(core_axis_name="c", subcore_axis_name="s")

  @functools.partial(
      pl.kernel,
      mesh=mesh,
      compiler_params=pltpu.CompilerParams(
          use_tc_tiling_on_sc=False, needs_layout_passes=False
      ),
      out_type=jax.ShapeDtypeStruct((H, NT, D), jnp.float32),
      scratch_types=[
          pltpu.VMEM((TPW, H), jnp.int32),
          pltpu.VMEM((H, NQ, CH), jnp.int32),
          pltpu.VMEM((TPW, D), jnp.float32),
          pltpu.VMEM((TPW, D), jnp.float32),
          pltpu.SemaphoreType.DMA,
          pltpu.SemaphoreType.DMA,
          pltpu.SemaphoreType.DMA,
          pltpu.SemaphoreType.DMA,
      ],
  )
  def gather_kernel(tokens_hbm, table_hbm, out_hbm, idx_v, idx_t,
                    buf0, buf1, gsem0, gsem1, ssem0, ssem1):
    wid = lax.axis_index("s") * NC + lax.axis_index("c")
    b0 = wid * TPW
    bufs = (buf0, buf1)
    gsems = (gsem0, gsem1)
    ssems = (ssem0, ssem1)
    lanes = lax.iota(jnp.int32, 16)

    pltpu.sync_copy(tokens_hbm.at[pl.ds(b0, TPW)], idx_v)

    def idx_rows(h):
      for g in range(TPW // 16):
        v = plsc.load_gather(
            idx_v, [g * 16 + lanes, jnp.full((16,), h, jnp.int32)]
        )
        idx_t[h, g // 8, pl.ds((g % 8) * 16, 16)] = v
    pl.loop(0, H)(idx_rows)

    def fire(h, buf, gsem):
      for q in range(NQ):
        pltpu.make_async_copy(
            table_hbm.at[idx_t.at[h, q]],
            buf.at[pl.ds(q * CH, CH)],
            gsem,
        ).start()

    def drain_gathers(b):
      pltpu.make_async_copy(
          table_hbm.at[pl.ds(0, TPW)], bufs[b], gsems[b]
      ).wait()

    def pair(p):
      for b in range(2):
        h = p + b
        ob = 1 - b

        @pl.when(h >= 2)
        def _():
          pltpu.make_async_copy(
              bufs[b], out_hbm.at[0, pl.ds(0, TPW)], ssems[b]
          ).wait()

        fire(h, bufs[b], gsems[b])

        @pl.when(h >= 1)
        def _():
          drain_gathers(ob)
          pltpu.make_async_copy(
              bufs[ob], out_hbm.at[h - 1, pl.ds(b0, TPW)], ssems[ob]
          ).start()

    pl.loop(0, H, step=2)(pair)

    drain_gathers(1)
    final = pltpu.make_async_copy(
        bufs[1], out_hbm.at[H - 1, pl.ds(b0, TPW)], ssems[1]
    )
    final.start()
    pltpu.make_async_copy(
        bufs[0], out_hbm.at[0, pl.ds(0, TPW)], ssems[0]
    ).wait()
    final.wait()

  return gather_kernel


_gather = _make_gather()


def kernel(tokens, table):
  out_hmaj = _gather(tokens, table)
  return out_hmaj.transpose(1, 0, 2)
